# Initial kernel scaffold; baseline (speedup 1.0000x reference)
import functools
import math

import jax
import jax.numpy as jnp
from jax import lax
from jax.experimental import pallas as pl
from jax.experimental.pallas import tpu as pltpu
from jax.experimental.pallas import tpu_sc as plsc

EMB_D = 32
CHUNK = 128
SCALE = math.sqrt(EMB_D)


@functools.cache
def _build(num_chunks_total: int):
    info = plsc.get_sparse_core_info()
    nc, ns = info.num_cores, info.num_subcores
    nw = nc * ns
    assert num_chunks_total % nw == 0
    chunks_per_w = num_chunks_total // nw
    mesh = plsc.VectorSubcoreMesh(core_axis_name="c", subcore_axis_name="s")

    @functools.partial(
        pl.kernel,
        mesh=mesh,
        out_type=jax.ShapeDtypeStruct((num_chunks_total * CHUNK, EMB_D),
                                      jnp.float32),
        scratch_types=[
            pltpu.VMEM((CHUNK,), jnp.int32),
            pltpu.VMEM((CHUNK, EMB_D), jnp.float32),
            pltpu.SemaphoreType.DMA,
        ],
    )
    def gather_scale(tok_hbm, w_hbm, out_hbm, idx_v, rows_v, sem):
        wid = lax.axis_index("s") * nc + lax.axis_index("c")
        base_chunk = wid * chunks_per_w

        def chunk_body(j, carry):
            row = base_chunk + j
            pltpu.sync_copy(tok_hbm.at[row], idx_v)
            pltpu.async_copy(w_hbm.at[idx_v], rows_v, sem).wait()

            def scale_body(i, c):
                v0 = rows_v[i, pl.ds(0, 16)] * SCALE
                rows_v[i, pl.ds(0, 16)] = v0
                v1 = rows_v[i, pl.ds(16, 16)] * SCALE
                rows_v[i, pl.ds(16, 16)] = v1
                return c

            lax.fori_loop(0, CHUNK, scale_body, 0)
            pltpu.sync_copy(rows_v, out_hbm.at[pl.ds(row * CHUNK, CHUNK)])
            return carry

        lax.fori_loop(0, chunks_per_w, chunk_body, 0)

    return gather_scale


def kernel(tokens, weight):
    b, s = tokens.shape
    total = b * s
    assert total % CHUNK == 0
    tok2d = tokens.reshape(total // CHUNK, CHUNK).astype(jnp.int32)
    out = _build(total // CHUNK)(tok2d, weight)
    return out.reshape(b, s, EMB_D)


# speedup vs baseline: 1.0999x; 1.0999x over previous
import functools
import math

import jax
import jax.numpy as jnp
from jax import lax
from jax.experimental import pallas as pl
from jax.experimental.pallas import tpu as pltpu
from jax.experimental.pallas import tpu_sc as plsc

EMB_D = 32
CHUNK = 128
SCALE = math.sqrt(EMB_D)


@functools.cache
def _build(num_chunks_total: int):
    info = plsc.get_sparse_core_info()
    nc, ns = info.num_cores, info.num_subcores
    nw = nc * ns
    assert num_chunks_total % nw == 0
    chunks_per_w = num_chunks_total // nw
    mesh = plsc.VectorSubcoreMesh(core_axis_name="c", subcore_axis_name="s")

    @functools.partial(
        pl.kernel,
        mesh=mesh,
        out_type=jax.ShapeDtypeStruct((num_chunks_total * CHUNK, EMB_D),
                                      jnp.float32),
        scratch_types=[
            pltpu.VMEM((CHUNK,), jnp.int32),
            pltpu.VMEM((CHUNK, EMB_D), jnp.float32),
            pltpu.SemaphoreType.DMA,
        ],
        compiler_params=pltpu.CompilerParams(use_tc_tiling_on_sc=False),
    )
    def gather_scale(tok_hbm, w_hbm, out_hbm, idx_v, rows_v, sem):
        wid = lax.axis_index("s") * nc + lax.axis_index("c")
        base_chunk = wid * chunks_per_w

        def chunk_body(j, carry):
            row = base_chunk + j
            pltpu.sync_copy(tok_hbm.at[row], idx_v)
            pltpu.async_copy(w_hbm.at[idx_v], rows_v, sem).wait()

            def scale_body(i, c):
                v0 = rows_v[i, pl.ds(0, 16)] * SCALE
                rows_v[i, pl.ds(0, 16)] = v0
                v1 = rows_v[i, pl.ds(16, 16)] * SCALE
                rows_v[i, pl.ds(16, 16)] = v1
                return c

            lax.fori_loop(0, CHUNK, scale_body, 0)
            pltpu.sync_copy(rows_v, out_hbm.at[pl.ds(row * CHUNK, CHUNK)])
            return carry

        lax.fori_loop(0, chunks_per_w, chunk_body, 0)

    return gather_scale


def kernel(tokens, weight):
    b, s = tokens.shape
    total = b * s
    assert total % CHUNK == 0
    tok2d = tokens.reshape(total // CHUNK, CHUNK).astype(jnp.int32)
    out = _build(total // CHUNK)(tok2d, weight)
    return out.reshape(b, s, EMB_D)


# speedup vs baseline: 1.4759x; 1.3418x over previous
import functools
import math

import jax
import jax.numpy as jnp
from jax import lax
from jax.experimental import pallas as pl
from jax.experimental.pallas import tpu as pltpu
from jax.experimental.pallas import tpu_sc as plsc

EMB_D = 32
CHUNK = 128
GPM = 8
MACRO = CHUNK * GPM
SCALE = math.sqrt(EMB_D)


@functools.cache
def _build(num_chunks_total: int):
    info = plsc.get_sparse_core_info()
    nc, ns = info.num_cores, info.num_subcores
    nw = nc * ns
    assert num_chunks_total % (nw * GPM) == 0
    chunks_per_w = num_chunks_total // nw
    macros_per_w = chunks_per_w // GPM
    assert macros_per_w >= 4 and macros_per_w % 2 == 1
    mesh = plsc.VectorSubcoreMesh(core_axis_name="c", subcore_axis_name="s")

    @functools.partial(
        pl.kernel,
        mesh=mesh,
        out_type=jax.ShapeDtypeStruct((num_chunks_total * CHUNK, EMB_D),
                                      jnp.float32),
        scratch_types=[
            pltpu.VMEM((chunks_per_w, CHUNK), jnp.int32),
            pltpu.VMEM((MACRO, EMB_D), jnp.float32),
            pltpu.VMEM((MACRO, EMB_D), jnp.float32),
            pltpu.SemaphoreType.DMA,
            pltpu.SemaphoreType.DMA,
            pltpu.SemaphoreType.DMA,
            pltpu.SemaphoreType.DMA,
        ],
        compiler_params=pltpu.CompilerParams(use_tc_tiling_on_sc=False),
    )
    def gather_scale(tok_hbm, w_hbm, out_hbm, idx_v, rows0, rows1,
                     gsem0, gsem1, ssem0, ssem1):
        wid = lax.axis_index("s") * nc + lax.axis_index("c")
        base_chunk = wid * chunks_per_w
        rows = (rows0, rows1)
        gsems = (gsem0, gsem1)
        ssems = (ssem0, ssem1)

        pltpu.sync_copy(tok_hbm.at[pl.ds(base_chunk, chunks_per_w)], idx_v)

        def fire(m, b, drain_store):
            if drain_store:
                pltpu.make_async_copy(
                    rows[b], out_hbm.at[pl.ds(0, MACRO)], ssems[b]).wait()
            for g in range(GPM):
                pltpu.async_copy(w_hbm.at[idx_v.at[m * GPM + g]],
                                 rows[b].at[pl.ds(g * CHUNK, CHUNK)],
                                 gsems[b])

        def proc(m, b):
            pltpu.make_async_copy(
                w_hbm.at[pl.ds(0, MACRO)], rows[b], gsems[b]).wait()
            rows_b = rows[b]

            @plsc.parallel_loop(0, MACRO, unroll=8)
            def _(r):
                rows_b[r, pl.ds(0, 16)] = rows_b[r, pl.ds(0, 16)] * SCALE
                rows_b[r, pl.ds(16, 16)] = rows_b[r, pl.ds(16, 16)] * SCALE

            out_row = (base_chunk + m * GPM) * CHUNK
            pltpu.async_copy(rows_b, out_hbm.at[pl.ds(out_row, MACRO)],
                             ssems[b])

        fire(0, 0, False)
        fire(1, 1, False)
        proc(0, 0)
        fire(2, 0, True)
        proc(1, 1)

        @pl.loop(2, macros_per_w - 1, step=2)
        def _(mp):
            fire(mp + 1, 1, True)
            proc(mp, 0)
            fire(mp + 2, 0, True)
            proc(mp + 1, 1)

        proc(macros_per_w - 1, 0)

        pltpu.make_async_copy(rows0, out_hbm.at[pl.ds(0, MACRO)], ssem0).wait()
        pltpu.make_async_copy(rows1, out_hbm.at[pl.ds(0, MACRO)], ssem1).wait()

    return gather_scale


def kernel(tokens, weight):
    b, s = tokens.shape
    total = b * s
    assert total % CHUNK == 0
    tok2d = tokens.reshape(total // CHUNK, CHUNK).astype(jnp.int32)
    out = _build(total // CHUNK)(tok2d, weight)
    return out.reshape(b, s, EMB_D)


# speedup vs baseline: 1.4791x; 1.0022x over previous
import functools
import math

import jax
import jax.numpy as jnp
from jax import lax
from jax.experimental import pallas as pl
from jax.experimental.pallas import tpu as pltpu
from jax.experimental.pallas import tpu_sc as plsc

EMB_D = 32
CHUNK = 128
GPM = 5
MACRO = CHUNK * GPM
NBUF = 4
AHEAD = 2
SCALE = math.sqrt(EMB_D)


@functools.cache
def _build(num_chunks_total: int):
    info = plsc.get_sparse_core_info()
    nc, ns = info.num_cores, info.num_subcores
    nw = nc * ns
    assert num_chunks_total % (nw * GPM) == 0
    chunks_per_w = num_chunks_total // nw
    M = chunks_per_w // GPM
    assert M % NBUF == 0 and M >= 2 * NBUF
    mesh = plsc.VectorSubcoreMesh(core_axis_name="c", subcore_axis_name="s")

    @functools.partial(
        pl.kernel,
        mesh=mesh,
        out_type=jax.ShapeDtypeStruct((num_chunks_total * CHUNK, EMB_D),
                                      jnp.float32),
        scratch_types=[
            pltpu.VMEM((chunks_per_w, CHUNK), jnp.int32),
            pltpu.VMEM((MACRO, EMB_D), jnp.float32),
            pltpu.VMEM((MACRO, EMB_D), jnp.float32),
            pltpu.VMEM((MACRO, EMB_D), jnp.float32),
            pltpu.VMEM((MACRO, EMB_D), jnp.float32),
            pltpu.SemaphoreType.DMA,
            pltpu.SemaphoreType.DMA,
            pltpu.SemaphoreType.DMA,
            pltpu.SemaphoreType.DMA,
            pltpu.SemaphoreType.DMA,
            pltpu.SemaphoreType.DMA,
            pltpu.SemaphoreType.DMA,
            pltpu.SemaphoreType.DMA,
        ],
        compiler_params=pltpu.CompilerParams(use_tc_tiling_on_sc=False),
    )
    def gather_scale(tok_hbm, w_hbm, out_hbm, idx_v,
                     rows0, rows1, rows2, rows3,
                     gsem0, gsem1, gsem2, gsem3,
                     ssem0, ssem1, ssem2, ssem3):
        wid = lax.axis_index("s") * nc + lax.axis_index("c")
        base_chunk = wid * chunks_per_w
        rows = (rows0, rows1, rows2, rows3)
        gsems = (gsem0, gsem1, gsem2, gsem3)
        ssems = (ssem0, ssem1, ssem2, ssem3)

        pltpu.sync_copy(tok_hbm.at[pl.ds(base_chunk, chunks_per_w)], idx_v)

        def fire(m, b, drain_store):
            if drain_store:
                pltpu.make_async_copy(
                    rows[b], out_hbm.at[pl.ds(0, MACRO)], ssems[b]).wait()
            for g in range(GPM):
                pltpu.async_copy(w_hbm.at[idx_v.at[m * GPM + g]],
                                 rows[b].at[pl.ds(g * CHUNK, CHUNK)],
                                 gsems[b])

        def proc(m, b):
            pltpu.make_async_copy(
                w_hbm.at[pl.ds(0, MACRO)], rows[b], gsems[b]).wait()
            rows_b = rows[b]

            @plsc.parallel_loop(0, MACRO, unroll=8)
            def _(r):
                rows_b[r, pl.ds(0, 16)] = rows_b[r, pl.ds(0, 16)] * SCALE
                rows_b[r, pl.ds(16, 16)] = rows_b[r, pl.ds(16, 16)] * SCALE

            out_row = (base_chunk + m * GPM) * CHUNK
            pltpu.async_copy(rows_b, out_hbm.at[pl.ds(out_row, MACRO)],
                             ssems[b])

        fire(0, 0, False)
        fire(1, 1, False)
        fire(2, 2, False)
        proc(0, 0)
        fire(3, 3, False)
        proc(1, 1)

        @pl.loop(NBUF - AHEAD, M - AHEAD, step=NBUF)
        def _(mp):
            for k in range(NBUF):
                m = mp + k
                fire(m + AHEAD, k % NBUF, True)
                proc(m, (NBUF - AHEAD + k) % NBUF)

        proc(M - 2, (M - 2) % NBUF)
        proc(M - 1, (M - 1) % NBUF)

        for b in range(NBUF):
            pltpu.make_async_copy(
                rows[b], out_hbm.at[pl.ds(0, MACRO)], ssems[b]).wait()

    return gather_scale


def kernel(tokens, weight):
    b, s = tokens.shape
    total = b * s
    assert total % CHUNK == 0
    tok2d = tokens.reshape(total // CHUNK, CHUNK).astype(jnp.int32)
    out = _build(total // CHUNK)(tok2d, weight)
    return out.reshape(b, s, EMB_D)
